# Initial kernel scaffold; baseline (speedup 1.0000x reference)
import jax
import jax.numpy as jnp
from jax.experimental import pallas as pl


def kernel(s, z, f, edge_index, params):
    raise NotImplementedError("write your pallas kernel here")


# speedup vs baseline: 1.0572x; 1.0572x over previous
import functools
import jax
import jax.numpy as jnp
from jax.experimental import pallas as pl

N_NODES = 10000
N_EDGES = 160000
DS = 128
DP = 128
H = 12
C = 16
PQ = 4
PV = 8
EPS = 1e-08


def _node_proj_body(s_ref, rt_ref, w_ref, b_ref, q_ref, k_ref, v_ref,
                    qp_ref, kp_ref, vp_ref):
    s = s_ref[...]
    y = jnp.dot(s, w_ref[...], preferred_element_type=jnp.float32)
    y = y + b_ref[...]
    q_ref[...] = y[:, 0:192]
    k_ref[...] = y[:, 192:384]
    v_ref[...] = y[:, 384:576]
    rt = rt_ref[...]
    for (base, width, out_ref) in ((576, 48, qp_ref), (720, 48, kp_ref),
                                   (864, 96, vp_ref)):
        l0 = y[:, base:base + width]
        l1 = y[:, base + width:base + 2 * width]
        l2 = y[:, base + 2 * width:base + 3 * width]
        for i in range(3):
            gi = (rt[:, 3 * i:3 * i + 1] * l0
                  + rt[:, 3 * i + 1:3 * i + 2] * l1
                  + rt[:, 3 * i + 2:3 * i + 3] * l2
                  + rt[:, 9 + i:10 + i])
            out_ref[:, i, :] = gi


def _node_proj(s, rt, w_cat, b_cat):
    n = s.shape[0]
    bn = 1000
    grid = (n // bn,)
    return pl.pallas_call(
        _node_proj_body,
        grid=grid,
        in_specs=[
            pl.BlockSpec((bn, DS), lambda i: (i, 0)),
            pl.BlockSpec((bn, 12), lambda i: (i, 0)),
            pl.BlockSpec((DS, 1152), lambda i: (0, 0)),
            pl.BlockSpec((1, 1152), lambda i: (0, 0)),
        ],
        out_specs=[
            pl.BlockSpec((bn, 192), lambda i: (i, 0)),
            pl.BlockSpec((bn, 192), lambda i: (i, 0)),
            pl.BlockSpec((bn, 192), lambda i: (i, 0)),
            pl.BlockSpec((bn, 3, 48), lambda i: (i, 0, 0)),
            pl.BlockSpec((bn, 3, 48), lambda i: (i, 0, 0)),
            pl.BlockSpec((bn, 3, 96), lambda i: (i, 0, 0)),
        ],
        out_shape=[
            jax.ShapeDtypeStruct((n, 192), jnp.float32),
            jax.ShapeDtypeStruct((n, 192), jnp.float32),
            jax.ShapeDtypeStruct((n, 192), jnp.float32),
            jax.ShapeDtypeStruct((n, 3, 48), jnp.float32),
            jax.ShapeDtypeStruct((n, 3, 48), jnp.float32),
            jax.ShapeDtypeStruct((n, 3, 96), jnp.float32),
        ],
    )(s, rt, w_cat, b_cat)


def _edge_proj_body(z_ref, w_ref, b_ref, bias_ref, pz_ref):
    y = jnp.dot(z_ref[...], w_ref[...], preferred_element_type=jnp.float32)
    y = y + b_ref[...]
    bias_ref[...] = y[:, 0:H]
    pz_ref[...] = y[:, H:H + 192]


def _edge_proj(z, w_cat, b_cat):
    e = z.shape[0]
    be = 4000
    return pl.pallas_call(
        _edge_proj_body,
        grid=(e // be,),
        in_specs=[
            pl.BlockSpec((be, DP), lambda i: (i, 0)),
            pl.BlockSpec((DP, H + 192), lambda i: (0, 0)),
            pl.BlockSpec((1, H + 192), lambda i: (0, 0)),
        ],
        out_specs=[
            pl.BlockSpec((be, H), lambda i: (i, 0)),
            pl.BlockSpec((be, 192), lambda i: (i, 0)),
        ],
        out_shape=[
            jax.ShapeDtypeStruct((e, H), jnp.float32),
            jax.ShapeDtypeStruct((e, 192), jnp.float32),
        ],
    )(z, w_cat, b_cat)


def _out_proj_body(x_ref, w_ref, b_ref, o_ref):
    o_ref[...] = (jnp.dot(x_ref[...], w_ref[...],
                          preferred_element_type=jnp.float32) + b_ref[...])


def _out_proj(x, w_t, b):
    n = x.shape[0]
    bn = 1000
    din = x.shape[1]
    return pl.pallas_call(
        _out_proj_body,
        grid=(n // bn,),
        in_specs=[
            pl.BlockSpec((bn, din), lambda i: (i, 0)),
            pl.BlockSpec((din, DS), lambda i: (0, 0)),
            pl.BlockSpec((1, DS), lambda i: (0, 0)),
        ],
        out_specs=pl.BlockSpec((bn, DS), lambda i: (i, 0)),
        out_shape=jax.ShapeDtypeStruct((n, DS), jnp.float32),
    )(x, w_t, b)


def _coord_major_rows(w, npts):
    w3 = w.reshape(H, npts, 3, DS)
    return w3.transpose(2, 0, 1, 3).reshape(H * npts * 3, DS)


def kernel(s, z, f, edge_index, params):
    p = params
    dst = edge_index[:, 0]
    src = edge_index[:, 1]
    rt = jnp.concatenate(
        [f[:, :3, :3].reshape(N_NODES, 9), f[:, :3, 3]], axis=-1)

    w_node = jnp.concatenate([
        p['Wq'], p['Wk'], p['Wv'],
        _coord_major_rows(p['Wqp'], PQ),
        _coord_major_rows(p['Wkp'], PQ),
        _coord_major_rows(p['Wvp'], PV),
    ], axis=0).T
    b_node = jnp.concatenate([
        p['bq'], p['bk'], p['bv'],
        p['bqp'].reshape(H, PQ, 3).transpose(2, 0, 1).reshape(-1),
        p['bkp'].reshape(H, PQ, 3).transpose(2, 0, 1).reshape(-1),
        p['bvp'].reshape(H, PV, 3).transpose(2, 0, 1).reshape(-1),
    ])[None, :]

    q, k, v, qpg, kpg, vpg = _node_proj(s, rt, w_node, b_node)
    qp = qpg.reshape(N_NODES, 3, H, PQ).transpose(0, 2, 3, 1)
    kp = kpg.reshape(N_NODES, 3, H, PQ).transpose(0, 2, 3, 1)
    vp = vpg.reshape(N_NODES, 3, H, PV).transpose(0, 2, 3, 1)
    q = q.reshape(N_NODES, H, C)
    k = k.reshape(N_NODES, H, C)
    v = v.reshape(N_NODES, H, C)

    w_edge = jnp.concatenate([p['Wb'], p['Wpair']], axis=0).T
    b_edge = jnp.concatenate([p['bb'], p['bpair']])[None, :]
    bias, pz = _edge_proj(z, w_edge, b_edge)

    norm_attn = (1.0 / (3 * C)) ** (-0.5)
    attn = jnp.sum(q[dst] * k[src], axis=-1) * norm_attn
    attn = attn + bias * (1.0 / 3) ** (-0.5)
    pt_att = jnp.sum((qp[dst] - kp[src]) ** 2, axis=(-1, -2))
    hw = jax.nn.softplus(p['head_weights'])
    norm_pt = -0.5 * (1.0 / (3 * (PQ * 9.0 / 2))) ** (-0.5)
    attn = attn + pt_att * hw * norm_pt
    seg_max = jax.ops.segment_max(attn, dst, num_segments=N_NODES)
    attn = jnp.exp(attn - seg_max[dst])
    denom = jax.ops.segment_sum(attn, dst, num_segments=N_NODES)
    attn = attn / denom[dst]

    o = jax.ops.segment_sum(attn[..., None] * v[src], dst,
                            num_segments=N_NODES).reshape(N_NODES, H * C)
    o_pts_g = jax.ops.segment_sum(attn[..., None, None] * vp[src], dst,
                                  num_segments=N_NODES)
    R = f[:, :3, :3]
    t = f[:, :3, 3]
    o_pts = jnp.einsum('nji,nhpj->nhpi', R,
                       o_pts_g - t[:, None, None, :])
    pt_norm = jnp.sqrt(jnp.sum(o_pts ** 2, axis=-1) + EPS)
    o_pair = jax.ops.segment_sum(attn[..., None] * pz.reshape(N_EDGES, H, C),
                                 dst, num_segments=N_NODES).reshape(
                                     N_NODES, H * C)
    cat = jnp.concatenate([o, o_pts.reshape(N_NODES, -1),
                           pt_norm.reshape(N_NODES, -1), o_pair], axis=-1)
    return _out_proj(cat, p['Wo'].T, p['bo'][None, :])


# speedup vs baseline: 5.6301x; 5.3257x over previous
import functools
import jax
import jax.numpy as jnp
from jax import lax
from jax.experimental import pallas as pl
from jax.experimental.pallas import tpu as pltpu
from jax.experimental.pallas import tpu_sc as plsc

N_NODES = 10000
N_EDGES = 160000
DS = 128
DP = 128
H = 12
C = 16
PQ = 4
PV = 8
EPS = 1e-08


def _node_proj_body(s_ref, rt_ref, w_ref, b_ref, q_ref, k_ref, v_ref,
                    qp_ref, kp_ref, vp_ref):
    s = s_ref[...]
    y = jnp.dot(s, w_ref[...], preferred_element_type=jnp.float32)
    y = y + b_ref[...]
    q_ref[...] = y[:, 0:192]
    k_ref[...] = y[:, 192:384]
    v_ref[...] = y[:, 384:576]
    rt = rt_ref[...]
    for (base, width, out_ref) in ((576, 48, qp_ref), (720, 48, kp_ref),
                                   (864, 96, vp_ref)):
        l0 = y[:, base:base + width]
        l1 = y[:, base + width:base + 2 * width]
        l2 = y[:, base + 2 * width:base + 3 * width]
        for i in range(3):
            gi = (rt[:, 3 * i:3 * i + 1] * l0
                  + rt[:, 3 * i + 1:3 * i + 2] * l1
                  + rt[:, 3 * i + 2:3 * i + 3] * l2
                  + rt[:, 9 + i:10 + i])
            out_ref[:, i, :] = gi


def _node_proj(s, rt, w_cat, b_cat):
    n = s.shape[0]
    bn = 1000
    grid = (n // bn,)
    return pl.pallas_call(
        _node_proj_body,
        grid=grid,
        in_specs=[
            pl.BlockSpec((bn, DS), lambda i: (i, 0)),
            pl.BlockSpec((bn, 12), lambda i: (i, 0)),
            pl.BlockSpec((DS, 1152), lambda i: (0, 0)),
            pl.BlockSpec((1, 1152), lambda i: (0, 0)),
        ],
        out_specs=[
            pl.BlockSpec((bn, 192), lambda i: (i, 0)),
            pl.BlockSpec((bn, 192), lambda i: (i, 0)),
            pl.BlockSpec((bn, 192), lambda i: (i, 0)),
            pl.BlockSpec((bn, 3, 48), lambda i: (i, 0, 0)),
            pl.BlockSpec((bn, 3, 48), lambda i: (i, 0, 0)),
            pl.BlockSpec((bn, 3, 96), lambda i: (i, 0, 0)),
        ],
        out_shape=[
            jax.ShapeDtypeStruct((n, 192), jnp.float32),
            jax.ShapeDtypeStruct((n, 192), jnp.float32),
            jax.ShapeDtypeStruct((n, 192), jnp.float32),
            jax.ShapeDtypeStruct((n, 3, 48), jnp.float32),
            jax.ShapeDtypeStruct((n, 3, 48), jnp.float32),
            jax.ShapeDtypeStruct((n, 3, 96), jnp.float32),
        ],
    )(s, rt, w_cat, b_cat)


def _edge_proj_body(z_ref, w_ref, b_ref, bias_ref, pz_ref):
    y = jnp.dot(z_ref[...], w_ref[...], preferred_element_type=jnp.float32)
    y = y + b_ref[...]
    bias_ref[...] = y[:, 0:H]
    pz_ref[...] = y[:, H:H + 192]


def _edge_proj(z, w_cat, b_cat):
    e = z.shape[0]
    be = 4000
    return pl.pallas_call(
        _edge_proj_body,
        grid=(e // be,),
        in_specs=[
            pl.BlockSpec((be, DP), lambda i: (i, 0)),
            pl.BlockSpec((DP, H + 192), lambda i: (0, 0)),
            pl.BlockSpec((1, H + 192), lambda i: (0, 0)),
        ],
        out_specs=[
            pl.BlockSpec((be, H), lambda i: (i, 0)),
            pl.BlockSpec((be, 192), lambda i: (i, 0)),
        ],
        out_shape=[
            jax.ShapeDtypeStruct((e, H), jnp.float32),
            jax.ShapeDtypeStruct((e, 192), jnp.float32),
        ],
    )(z, w_cat, b_cat)


def _out_proj_body(x_ref, w_ref, b_ref, o_ref):
    o_ref[...] = (jnp.dot(x_ref[...], w_ref[...],
                          preferred_element_type=jnp.float32) + b_ref[...])


def _out_proj(x, w_t, b):
    n = x.shape[0]
    bn = 1000
    din = x.shape[1]
    return pl.pallas_call(
        _out_proj_body,
        grid=(n // bn,),
        in_specs=[
            pl.BlockSpec((bn, din), lambda i: (i, 0)),
            pl.BlockSpec((din, DS), lambda i: (0, 0)),
            pl.BlockSpec((1, DS), lambda i: (0, 0)),
        ],
        out_specs=pl.BlockSpec((bn, DS), lambda i: (i, 0)),
        out_shape=jax.ShapeDtypeStruct((n, DS), jnp.float32),
    )(x, w_t, b)


E_PAD = 163840
V_CHUNK = 128
N_CHUNKS = E_PAD // (32 * V_CHUNK)
N_ACC = 10240
NPT = N_ACC // 16
VW = 96


def _value_pass(table, gidx2d, dst2d, wexp, zeros):
    mesh = plsc.VectorSubcoreMesh(core_axis_name="c", subcore_axis_name="s")

    def body(table_hbm, gidx_hbm, dst_hbm, w_hbm, z_hbm, out_hbm,
             gidx_v, didx_v, w_v, rows_v, prod_v, acc):
        c = lax.axis_index("c")
        s = lax.axis_index("s")
        wid = c * 16 + s
        pltpu.sync_copy(z_hbm.at[pl.ds(s * NPT, NPT)],
                        acc.at[pl.ds(s * NPT, NPT)])
        pltpu.sync_copy(gidx_hbm.at[pl.ds(wid * N_CHUNKS, N_CHUNKS)], gidx_v)
        pltpu.sync_copy(dst_hbm.at[pl.ds(wid * N_CHUNKS, N_CHUNKS)], didx_v)
        plsc.subcore_barrier()

        def chunk_body(ch, carry):
            e0 = wid * (N_CHUNKS * V_CHUNK) + ch * V_CHUNK
            pltpu.sync_copy(w_hbm.at[pl.ds(e0, V_CHUNK)], w_v)
            pltpu.sync_copy(table_hbm.at[gidx_v.at[ch]], rows_v)

            def edge_body(e, carry2):
                for cg in range(VW // 16):
                    prod_v[e, pl.ds(cg * 16, 16)] = (
                        w_v[e, pl.ds(cg * 16, 16)]
                        * rows_v[e, pl.ds(cg * 16, 16)])
                return carry2

            lax.fori_loop(0, V_CHUNK, edge_body, 0)
            pltpu.sync_copy(prod_v, acc.at[didx_v.at[ch]], add=True)
            return carry

        lax.fori_loop(0, N_CHUNKS, chunk_body, 0)
        plsc.subcore_barrier()
        pltpu.sync_copy(acc.at[pl.ds(s * NPT, NPT)],
                        out_hbm.at[c, pl.ds(s * NPT, NPT)])

    kfn = pl.kernel(
        body,
        mesh=mesh,
        compiler_params=pltpu.CompilerParams(use_tc_tiling_on_sc=False),
        out_type=jax.ShapeDtypeStruct((2, N_ACC, VW), jnp.float32),
        scratch_types=[
            pltpu.VMEM((N_CHUNKS, V_CHUNK), jnp.int32),
            pltpu.VMEM((N_CHUNKS, V_CHUNK), jnp.int32),
            pltpu.VMEM((V_CHUNK, VW), jnp.float32),
            pltpu.VMEM((V_CHUNK, VW), jnp.float32),
            pltpu.VMEM((V_CHUNK, VW), jnp.float32),
            pltpu.VMEM_SHARED((N_ACC, VW), jnp.float32),
        ],
    )
    out = kfn(table, gidx2d, dst2d, wexp, zeros)
    return (out[0] + out[1])[:N_NODES]


def _coord_major_rows(w, npts):
    w3 = w.reshape(H, npts, 3, DS)
    return w3.transpose(2, 0, 1, 3).reshape(H * npts * 3, DS)


def kernel(s, z, f, edge_index, params):
    p = params
    dst = edge_index[:, 0]
    src = edge_index[:, 1]
    rt = jnp.concatenate(
        [f[:, :3, :3].reshape(N_NODES, 9), f[:, :3, 3]], axis=-1)

    w_node = jnp.concatenate([
        p['Wq'], p['Wk'], p['Wv'],
        _coord_major_rows(p['Wqp'], PQ),
        _coord_major_rows(p['Wkp'], PQ),
        _coord_major_rows(p['Wvp'], PV),
    ], axis=0).T
    b_node = jnp.concatenate([
        p['bq'], p['bk'], p['bv'],
        p['bqp'].reshape(H, PQ, 3).transpose(2, 0, 1).reshape(-1),
        p['bkp'].reshape(H, PQ, 3).transpose(2, 0, 1).reshape(-1),
        p['bvp'].reshape(H, PV, 3).transpose(2, 0, 1).reshape(-1),
    ])[None, :]

    q, k, v192, qpg, kpg, vpg = _node_proj(s, rt, w_node, b_node)
    qp = qpg.reshape(N_NODES, 3, H, PQ).transpose(0, 2, 3, 1)
    kp = kpg.reshape(N_NODES, 3, H, PQ).transpose(0, 2, 3, 1)
    q = q.reshape(N_NODES, H, C)
    k = k.reshape(N_NODES, H, C)

    w_edge = jnp.concatenate([p['Wb'], p['Wpair']], axis=0).T
    b_edge = jnp.concatenate([p['bb'], p['bpair']])[None, :]
    bias, pz = _edge_proj(z, w_edge, b_edge)

    norm_attn = (1.0 / (3 * C)) ** (-0.5)
    attn = jnp.sum(q[dst] * k[src], axis=-1) * norm_attn
    attn = attn + bias * (1.0 / 3) ** (-0.5)
    pt_att = jnp.sum((qp[dst] - kp[src]) ** 2, axis=(-1, -2))
    hw = jax.nn.softplus(p['head_weights'])
    norm_pt = -0.5 * (1.0 / (3 * (PQ * 9.0 / 2))) ** (-0.5)
    attn = attn + pt_att * hw * norm_pt
    seg_max = jax.ops.segment_max(attn, dst, num_segments=N_NODES)
    attn = jnp.exp(attn - seg_max[dst])
    denom = jax.ops.segment_sum(attn, dst, num_segments=N_NODES)
    attn = attn / denom[dst]

    pad_e = E_PAD - N_EDGES
    dst_pad = jnp.pad(dst, (0, pad_e)).reshape(E_PAD // V_CHUNK, V_CHUNK)
    src_pad = jnp.pad(src, (0, pad_e)).reshape(E_PAD // V_CHUNK, V_CHUNK)
    eid_pad = jnp.minimum(jnp.arange(E_PAD, dtype=jnp.int32),
                          N_EDGES - 1).reshape(E_PAD // V_CHUNK, V_CHUNK)
    zeros = jnp.zeros((N_ACC, VW), jnp.float32)

    wv0 = jnp.pad(attn[:, jnp.arange(96) // 16], ((0, pad_e), (0, 0)))
    wv1 = jnp.pad(attn[:, 6 + jnp.arange(96) // 16], ((0, pad_e), (0, 0)))
    wvp = jnp.pad(attn[:, jnp.arange(96) // 8], ((0, pad_e), (0, 0)))

    o0 = _value_pass(v192[:, :96], src_pad, dst_pad, wv0, zeros)
    o1 = _value_pass(v192[:, 96:], src_pad, dst_pad, wv1, zeros)
    p0 = _value_pass(pz[:, :96], eid_pad, dst_pad, wv0, zeros)
    p1 = _value_pass(pz[:, 96:], eid_pad, dst_pad, wv1, zeros)
    vp_num = [
        _value_pass(vpg[:, i, :], src_pad, dst_pad, wvp, zeros)
        for i in range(3)
    ]
    o = jnp.concatenate([o0, o1], axis=-1)
    o_pair = jnp.concatenate([p0, p1], axis=-1)

    o_pts_g = jnp.stack(vp_num, axis=1)
    o_pts_g = o_pts_g.reshape(N_NODES, 3, H, PV)
    o_pts_g = o_pts_g.transpose(0, 2, 3, 1)
    R = f[:, :3, :3]
    t = f[:, :3, 3]
    o_pts = jnp.einsum('nji,nhpj->nhpi', R,
                       o_pts_g - t[:, None, None, :])
    pt_norm = jnp.sqrt(jnp.sum(o_pts ** 2, axis=-1) + EPS)
    cat = jnp.concatenate([o, o_pts.reshape(N_NODES, -1),
                           pt_norm.reshape(N_NODES, -1), o_pair], axis=-1)
    return _out_proj(cat, p['Wo'].T, p['bo'][None, :])


# speedup vs baseline: 7.9687x; 1.4154x over previous
import functools
import jax
import jax.numpy as jnp
from jax import lax
from jax.experimental import pallas as pl
from jax.experimental.pallas import tpu as pltpu
from jax.experimental.pallas import tpu_sc as plsc

N_NODES = 10000
N_EDGES = 160000
DS = 128
DP = 128
H = 12
C = 16
PQ = 4
PV = 8
EPS = 1e-08


def _node_proj_body(s_ref, rt_ref, w_ref, b_ref, q_ref, k_ref, v_ref,
                    qp_ref, kp_ref, vp_ref):
    s = s_ref[...]
    y = jnp.dot(s, w_ref[...], preferred_element_type=jnp.float32)
    y = y + b_ref[...]
    q_ref[...] = y[:, 0:192]
    k_ref[...] = y[:, 192:384]
    v_ref[...] = y[:, 384:576]
    rt = rt_ref[...]
    for (base, width, out_ref) in ((576, 48, qp_ref), (720, 48, kp_ref),
                                   (864, 96, vp_ref)):
        l0 = y[:, base:base + width]
        l1 = y[:, base + width:base + 2 * width]
        l2 = y[:, base + 2 * width:base + 3 * width]
        for i in range(3):
            gi = (rt[:, 3 * i:3 * i + 1] * l0
                  + rt[:, 3 * i + 1:3 * i + 2] * l1
                  + rt[:, 3 * i + 2:3 * i + 3] * l2
                  + rt[:, 9 + i:10 + i])
            out_ref[:, i, :] = gi


def _node_proj(s, rt, w_cat, b_cat):
    n = s.shape[0]
    bn = 1000
    grid = (n // bn,)
    return pl.pallas_call(
        _node_proj_body,
        grid=grid,
        in_specs=[
            pl.BlockSpec((bn, DS), lambda i: (i, 0)),
            pl.BlockSpec((bn, 12), lambda i: (i, 0)),
            pl.BlockSpec((DS, 1152), lambda i: (0, 0)),
            pl.BlockSpec((1, 1152), lambda i: (0, 0)),
        ],
        out_specs=[
            pl.BlockSpec((bn, 192), lambda i: (i, 0)),
            pl.BlockSpec((bn, 192), lambda i: (i, 0)),
            pl.BlockSpec((bn, 192), lambda i: (i, 0)),
            pl.BlockSpec((bn, 3, 48), lambda i: (i, 0, 0)),
            pl.BlockSpec((bn, 3, 48), lambda i: (i, 0, 0)),
            pl.BlockSpec((bn, 3, 96), lambda i: (i, 0, 0)),
        ],
        out_shape=[
            jax.ShapeDtypeStruct((n, 192), jnp.float32),
            jax.ShapeDtypeStruct((n, 192), jnp.float32),
            jax.ShapeDtypeStruct((n, 192), jnp.float32),
            jax.ShapeDtypeStruct((n, 3, 48), jnp.float32),
            jax.ShapeDtypeStruct((n, 3, 48), jnp.float32),
            jax.ShapeDtypeStruct((n, 3, 96), jnp.float32),
        ],
    )(s, rt, w_cat, b_cat)


def _edge_proj_body(z_ref, w_ref, b_ref, bias_ref, pz_ref):
    y = jnp.dot(z_ref[...], w_ref[...], preferred_element_type=jnp.float32)
    y = y + b_ref[...]
    bias_ref[...] = y[:, 0:H]
    pz_ref[...] = y[:, H:H + 192]


def _edge_proj(z, w_cat, b_cat):
    e = z.shape[0]
    be = 4000
    return pl.pallas_call(
        _edge_proj_body,
        grid=(e // be,),
        in_specs=[
            pl.BlockSpec((be, DP), lambda i: (i, 0)),
            pl.BlockSpec((DP, H + 192), lambda i: (0, 0)),
            pl.BlockSpec((1, H + 192), lambda i: (0, 0)),
        ],
        out_specs=[
            pl.BlockSpec((be, H), lambda i: (i, 0)),
            pl.BlockSpec((be, 192), lambda i: (i, 0)),
        ],
        out_shape=[
            jax.ShapeDtypeStruct((e, H), jnp.float32),
            jax.ShapeDtypeStruct((e, 192), jnp.float32),
        ],
    )(z, w_cat, b_cat)


def _out_proj_body(x_ref, w_ref, b_ref, o_ref):
    o_ref[...] = (jnp.dot(x_ref[...], w_ref[...],
                          preferred_element_type=jnp.float32) + b_ref[...])


def _out_proj(x, w_t, b):
    n = x.shape[0]
    bn = 1000
    din = x.shape[1]
    return pl.pallas_call(
        _out_proj_body,
        grid=(n // bn,),
        in_specs=[
            pl.BlockSpec((bn, din), lambda i: (i, 0)),
            pl.BlockSpec((din, DS), lambda i: (0, 0)),
            pl.BlockSpec((1, DS), lambda i: (0, 0)),
        ],
        out_specs=pl.BlockSpec((bn, DS), lambda i: (i, 0)),
        out_shape=jax.ShapeDtypeStruct((n, DS), jnp.float32),
    )(x, w_t, b)


E_PAD = 163840
V_CHUNK = 128
N_CHUNKS = E_PAD // (32 * V_CHUNK)
N_ACC = 10240
NPT = N_ACC // 16
VW = 96


def _value_pass(table, gidx2d, dst2d, wexp, zeros):
    mesh = plsc.VectorSubcoreMesh(core_axis_name="c", subcore_axis_name="s")

    def body(table_hbm, gidx_hbm, dst_hbm, w_hbm, z_hbm, out_hbm,
             gidx_v, didx_v, w_v, rows_v, prod_v, acc):
        c = lax.axis_index("c")
        s = lax.axis_index("s")
        wid = c * 16 + s
        pltpu.sync_copy(z_hbm.at[pl.ds(s * NPT, NPT)],
                        acc.at[pl.ds(s * NPT, NPT)])
        pltpu.sync_copy(gidx_hbm.at[pl.ds(wid * N_CHUNKS, N_CHUNKS)], gidx_v)
        pltpu.sync_copy(dst_hbm.at[pl.ds(wid * N_CHUNKS, N_CHUNKS)], didx_v)
        plsc.subcore_barrier()

        def chunk_body(ch, carry):
            e0 = wid * (N_CHUNKS * V_CHUNK) + ch * V_CHUNK
            pltpu.sync_copy(w_hbm.at[pl.ds(e0, V_CHUNK)], w_v)
            pltpu.sync_copy(table_hbm.at[gidx_v.at[ch]], rows_v)

            def edge_body(e, carry2):
                for cg in range(VW // 16):
                    prod_v[e, pl.ds(cg * 16, 16)] = (
                        w_v[e, pl.ds(cg * 16, 16)]
                        * rows_v[e, pl.ds(cg * 16, 16)])
                return carry2

            lax.fori_loop(0, V_CHUNK, edge_body, 0)
            pltpu.sync_copy(prod_v, acc.at[didx_v.at[ch]], add=True)
            return carry

        lax.fori_loop(0, N_CHUNKS, chunk_body, 0)
        plsc.subcore_barrier()
        pltpu.sync_copy(acc.at[pl.ds(s * NPT, NPT)],
                        out_hbm.at[c, pl.ds(s * NPT, NPT)])

    kfn = pl.kernel(
        body,
        mesh=mesh,
        compiler_params=pltpu.CompilerParams(use_tc_tiling_on_sc=False),
        out_type=jax.ShapeDtypeStruct((2, N_ACC, VW), jnp.float32),
        scratch_types=[
            pltpu.VMEM((N_CHUNKS, V_CHUNK), jnp.int32),
            pltpu.VMEM((N_CHUNKS, V_CHUNK), jnp.int32),
            pltpu.VMEM((V_CHUNK, VW), jnp.float32),
            pltpu.VMEM((V_CHUNK, VW), jnp.float32),
            pltpu.VMEM((V_CHUNK, VW), jnp.float32),
            pltpu.VMEM_SHARED((N_ACC, VW), jnp.float32),
        ],
    )
    out = kfn(table, gidx2d, dst2d, wexp, zeros)
    return (out[0] + out[1])[:N_NODES]


L_CHUNK = 64
L_CHUNKS = E_PAD // (32 * L_CHUNK)


def _logits_pass(td, ts, dst2d, src2d):
    mesh = plsc.VectorSubcoreMesh(core_axis_name="c", subcore_axis_name="s")

    def body(td_hbm, ts_hbm, dst_hbm, src_hbm, out_hbm,
             didx_v, sidx_v, td_v, ts_v, out_v):
        c = lax.axis_index("c")
        s = lax.axis_index("s")
        wid = c * 16 + s
        pltpu.sync_copy(dst_hbm.at[pl.ds(wid * L_CHUNKS, L_CHUNKS)], didx_v)
        pltpu.sync_copy(src_hbm.at[pl.ds(wid * L_CHUNKS, L_CHUNKS)], sidx_v)

        def chunk_body(ch, carry):
            e0 = wid * (L_CHUNKS * L_CHUNK) + ch * L_CHUNK
            pltpu.sync_copy(td_hbm.at[didx_v.at[ch]], td_v)
            pltpu.sync_copy(ts_hbm.at[sidx_v.at[ch]], ts_v)

            def edge_body(e, carry2):
                for h in range(12):
                    qv = td_v[e, pl.ds(h * 16, 16)]
                    kv = ts_v[e, pl.ds(h * 16, 16)]
                    qp = td_v[e, pl.ds(192 + h * 16, 16)]
                    kp = ts_v[e, pl.ds(192 + h * 16, 16)]
                    d = qp - kp
                    out_v[e, pl.ds(h * 16, 16)] = qv * kv - d * d
                return carry2

            lax.fori_loop(0, L_CHUNK, edge_body, 0)
            pltpu.sync_copy(out_v, out_hbm.at[pl.ds(e0, L_CHUNK)])
            return carry

        lax.fori_loop(0, L_CHUNKS, chunk_body, 0)

    kfn = pl.kernel(
        body,
        mesh=mesh,
        compiler_params=pltpu.CompilerParams(use_tc_tiling_on_sc=False),
        out_type=jax.ShapeDtypeStruct((E_PAD, 192), jnp.float32),
        scratch_types=[
            pltpu.VMEM((L_CHUNKS, L_CHUNK), jnp.int32),
            pltpu.VMEM((L_CHUNKS, L_CHUNK), jnp.int32),
            pltpu.VMEM((L_CHUNK, 384), jnp.float32),
            pltpu.VMEM((L_CHUNK, 384), jnp.float32),
            pltpu.VMEM((L_CHUNK, 192), jnp.float32),
        ],
    )
    return kfn(td, ts, dst2d, src2d)


def _coord_major_rows(w, npts):
    w3 = w.reshape(H, npts, 3, DS)
    return w3.transpose(2, 0, 1, 3).reshape(H * npts * 3, DS)


def kernel(s, z, f, edge_index, params):
    p = params
    dst = edge_index[:, 0]
    src = edge_index[:, 1]
    rt = jnp.concatenate(
        [f[:, :3, :3].reshape(N_NODES, 9), f[:, :3, 3]], axis=-1)

    w_node = jnp.concatenate([
        p['Wq'], p['Wk'], p['Wv'],
        _coord_major_rows(p['Wqp'], PQ),
        _coord_major_rows(p['Wkp'], PQ),
        _coord_major_rows(p['Wvp'], PV),
    ], axis=0).T
    b_node = jnp.concatenate([
        p['bq'], p['bk'], p['bv'],
        p['bqp'].reshape(H, PQ, 3).transpose(2, 0, 1).reshape(-1),
        p['bkp'].reshape(H, PQ, 3).transpose(2, 0, 1).reshape(-1),
        p['bvp'].reshape(H, PV, 3).transpose(2, 0, 1).reshape(-1),
    ])[None, :]

    q, k, v192, qpg, kpg, vpg = _node_proj(s, rt, w_node, b_node)

    w_edge = jnp.concatenate([p['Wb'], p['Wpair']], axis=0).T
    b_edge = jnp.concatenate([p['bb'], p['bpair']])[None, :]
    bias, pz = _edge_proj(z, w_edge, b_edge)

    pad_e = E_PAD - N_EDGES
    cn = (3.0 * C) ** 0.5
    hw = jax.nn.softplus(p['head_weights'])
    s_h = jnp.sqrt(hw * 0.5 * (3.0 * (PQ * 9.0 / 2)) ** 0.5)
    qp12 = qpg.reshape(N_NODES, 3, H, PQ).transpose(0, 2, 1, 3)
    qp12 = qp12.reshape(N_NODES, H, 12) * s_h[None, :, None]
    kp12 = kpg.reshape(N_NODES, 3, H, PQ).transpose(0, 2, 1, 3)
    kp12 = kp12.reshape(N_NODES, H, 12) * s_h[None, :, None]
    qp16 = jnp.pad(qp12, ((0, 0), (0, 0), (0, 4))).reshape(N_NODES, 192)
    kp16 = jnp.pad(kp12, ((0, 0), (0, 0), (0, 4))).reshape(N_NODES, 192)
    td_tab = jnp.concatenate([q * cn, qp16], axis=-1)
    ts_tab = jnp.concatenate([k, kp16], axis=-1)
    dst2d64 = jnp.pad(dst, (0, pad_e)).reshape(E_PAD // L_CHUNK, L_CHUNK)
    src2d64 = jnp.pad(src, (0, pad_e)).reshape(E_PAD // L_CHUNK, L_CHUNK)
    combo = _logits_pass(td_tab, ts_tab, dst2d64, src2d64)
    attn = (combo[:N_EDGES].reshape(N_EDGES, H, 16).sum(-1)
            + bias * (3.0 ** 0.5))
    seg_max = jax.ops.segment_max(attn, dst, num_segments=N_NODES)
    attn = jnp.exp(attn - seg_max[dst])
    denom = jax.ops.segment_sum(attn, dst, num_segments=N_NODES)
    attn = attn / denom[dst]

    dst_pad = jnp.pad(dst, (0, pad_e)).reshape(E_PAD // V_CHUNK, V_CHUNK)
    src_pad = jnp.pad(src, (0, pad_e)).reshape(E_PAD // V_CHUNK, V_CHUNK)
    eid_pad = jnp.minimum(jnp.arange(E_PAD, dtype=jnp.int32),
                          N_EDGES - 1).reshape(E_PAD // V_CHUNK, V_CHUNK)
    zeros = jnp.zeros((N_ACC, VW), jnp.float32)

    wv0 = jnp.pad(attn[:, jnp.arange(96) // 16], ((0, pad_e), (0, 0)))
    wv1 = jnp.pad(attn[:, 6 + jnp.arange(96) // 16], ((0, pad_e), (0, 0)))
    wvp = jnp.pad(attn[:, jnp.arange(96) // 8], ((0, pad_e), (0, 0)))

    o0 = _value_pass(v192[:, :96], src_pad, dst_pad, wv0, zeros)
    o1 = _value_pass(v192[:, 96:], src_pad, dst_pad, wv1, zeros)
    p0 = _value_pass(pz[:, :96], eid_pad, dst_pad, wv0, zeros)
    p1 = _value_pass(pz[:, 96:], eid_pad, dst_pad, wv1, zeros)
    vp_num = [
        _value_pass(vpg[:, i, :], src_pad, dst_pad, wvp, zeros)
        for i in range(3)
    ]
    o = jnp.concatenate([o0, o1], axis=-1)
    o_pair = jnp.concatenate([p0, p1], axis=-1)

    o_pts_g = jnp.stack(vp_num, axis=1)
    o_pts_g = o_pts_g.reshape(N_NODES, 3, H, PV)
    o_pts_g = o_pts_g.transpose(0, 2, 3, 1)
    R = f[:, :3, :3]
    t = f[:, :3, 3]
    o_pts = jnp.einsum('nji,nhpj->nhpi', R,
                       o_pts_g - t[:, None, None, :])
    pt_norm = jnp.sqrt(jnp.sum(o_pts ** 2, axis=-1) + EPS)
    cat = jnp.concatenate([o, o_pts.reshape(N_NODES, -1),
                           pt_norm.reshape(N_NODES, -1), o_pair], axis=-1)
    return _out_proj(cat, p['Wo'].T, p['bo'][None, :])
